# baseline (speedup 1.0000x reference)
import functools

import jax
import jax.numpy as jnp
from jax import lax
from jax.experimental import pallas as pl
from jax.experimental.pallas import tpu as pltpu
from jax.experimental.pallas import tpu_sc as plsc

VOCAB = 1000000
DIM = 64
SCALE = 8.0
NC = 2
NS = 16
NW = NC * NS
LANES = 16

N_FULL_BLK = VOCAB // 128
TAIL = VOCAB - N_FULL_BLK * 128
PAIRS = (N_FULL_BLK // NW + 2) // 2


def _mesh():
    return plsc.VectorSubcoreMesh(
        core_axis_name="c", subcore_axis_name="s", num_cores=NC, num_subcores=NS
    )


def _iota16():
    return lax.iota(jnp.int32, LANES)


PAD = 129


def _transpose_scaled(src, dst, n_src_rows, n_src_cols, scale=None):
    rows = [_iota16() + (w * LANES) for w in range(n_src_cols // LANES)]

    @plsc.parallel_loop(0, n_src_rows, 1, unroll=4)
    def _(r):
        col = jnp.full((LANES,), 0, jnp.int32) + r
        for w in range(n_src_cols // LANES):
            vals = src[r, pl.ds(w * LANES, LANES)]
            if scale is not None:
                vals = vals * scale
            plsc.store_scatter(dst, [rows[w], col], vals)


@functools.lru_cache(maxsize=None)
def _build_relayout():
    @functools.partial(
        pl.kernel,
        out_type=jax.ShapeDtypeStruct((VOCAB, 128), jnp.float32),
        mesh=_mesh(),
        scratch_types=[
            pltpu.VMEM((DIM, 128), jnp.float32),
            pltpu.VMEM((DIM, 128), jnp.float32),
            pltpu.VMEM((128, PAD), jnp.float32),
            pltpu.VMEM((128, PAD), jnp.float32),
            pltpu.SemaphoreType.DMA,
            pltpu.SemaphoreType.DMA,
            pltpu.SemaphoreType.DMA,
            pltpu.SemaphoreType.DMA,
        ],
        compiler_params=pltpu.CompilerParams(needs_layout_passes=False),
    )
    def relayout(tabT, tailp, tab2, gb0, gb1, tb0, tb1, si0, si1, so0, so1):
        wid = lax.axis_index("s") * NC + lax.axis_index("c")
        gbufs, tbufs = (gb0, gb1), (tb0, tb1)
        sins, souts = (si0, si1), (so0, so1)

        def blk(t):
            return t * NW + wid

        def start_in(t, p):
            @pl.when(blk(t) < N_FULL_BLK)
            def _():
                pltpu.async_copy(
                    tabT.at[:, pl.ds(blk(t) * 128, 128)], gbufs[p], sins[p]
                )

        start_in(0, 0)

        def pair(t2, c):
            for p in (0, 1):
                t = t2 * 2 + p
                g = blk(t)
                start_in(t + 1, 1 - p)

                @pl.when(g < N_FULL_BLK)
                def _():
                    pltpu.make_async_copy(
                        tabT.at[:, pl.ds(g * 128, 128)], gbufs[p], sins[p]
                    ).wait()

                    @pl.when(t >= 2)
                    def _w():
                        pltpu.make_async_copy(
                            tbufs[p].at[:, pl.ds(0, 128)],
                            tab2.at[pl.ds(0, 128), :],
                            souts[p],
                        ).wait()

                    _transpose_scaled(gbufs[p], tbufs[p], DIM, 128)
                    pltpu.async_copy(
                        tbufs[p].at[:, pl.ds(0, 128)],
                        tab2.at[pl.ds(g * 128, 128), :],
                        souts[p],
                    )

            return c

        lax.fori_loop(0, PAIRS, pair, 0)
        for p in (0, 1):
            pltpu.make_async_copy(
                tbufs[p].at[:, pl.ds(0, 128)], tab2.at[pl.ds(0, 128), :], souts[p]
            ).wait()

        @pl.when(wid == NW - 1)
        def _tail():
            v0 = N_FULL_BLK * 128
            pltpu.sync_copy(tailp, gb0)
            _transpose_scaled(gb0, tb0, DIM, TAIL)
            pltpu.sync_copy(
                tb0.at[pl.ds(0, TAIL), pl.ds(0, 128)], tab2.at[pl.ds(v0, TAIL), :]
            )

    return relayout


N_SBLK = 200 // 8
N_BBLK = 4096 // 128
UNITS_PER_W = N_SBLK * N_BBLK // NW


@functools.lru_cache(maxsize=None)
def _build_gather():
    @functools.partial(
        pl.kernel,
        out_type=jax.ShapeDtypeStruct((200, DIM, 4096), jnp.float32),
        mesh=_mesh(),
        scratch_types=[
            pltpu.VMEM((8, 128), jnp.int32),
            pltpu.VMEM((128, 128), jnp.float32),
            pltpu.VMEM((128, 128), jnp.float32),
            pltpu.VMEM((DIM, PAD), jnp.float32),
            pltpu.VMEM((DIM, PAD), jnp.float32),
            pltpu.SemaphoreType.DMA,
            pltpu.SemaphoreType.DMA,
            pltpu.SemaphoreType.DMA,
            pltpu.SemaphoreType.DMA,
        ],
        compiler_params=pltpu.CompilerParams(needs_layout_passes=False),
    )
    def gather(valsT, tab2, out3, idx_v, gb0, gb1, tr0, tr1, g0, g1, o0, o1):
        wid = lax.axis_index("s") * NC + lax.axis_index("c")
        gbufs, trbufs = (gb0, gb1), (tr0, tr1)
        gsems, osems = (g0, g1), (o0, o1)

        def do_unit(u, c):
            g = wid * UNITS_PER_W + u
            sb = g // N_BBLK
            bb = g - sb * N_BBLK
            s0 = sb * 8
            b0 = bb * 128
            pltpu.sync_copy(valsT.at[pl.ds(s0, 8), pl.ds(b0, 128)], idx_v)

            pltpu.async_copy(tab2.at[idx_v.at[0]], gbufs[0], gsems[0])
            for j in range(8):
                p = j % 2
                if j + 1 < 8:
                    pltpu.async_copy(
                        tab2.at[idx_v.at[j + 1]], gbufs[1 - p], gsems[1 - p]
                    )
                pltpu.make_async_copy(
                    tab2.at[idx_v.at[j]], gbufs[p], gsems[p]
                ).wait()
                if j >= 2:
                    pltpu.make_async_copy(
                        trbufs[p].at[:, pl.ds(0, 128)],
                        out3.at[s0, :, pl.ds(b0, 128)],
                        osems[p],
                    ).wait()

                _transpose_scaled(gbufs[p], trbufs[p], 128, DIM, SCALE)

                pltpu.async_copy(
                    trbufs[p].at[:, pl.ds(0, 128)],
                    out3.at[s0 + j, :, pl.ds(b0, 128)],
                    osems[p],
                )

            for p in (0, 1):
                pltpu.make_async_copy(
                    trbufs[p].at[:, pl.ds(0, 128)],
                    out3.at[s0, :, pl.ds(b0, 128)],
                    osems[p],
                ).wait()
            return c

        lax.fori_loop(0, UNITS_PER_W, do_unit, 0)

    return gather


def kernel(values, table):
    tabT = table.T
    valsT = values.T
    tailp = jnp.pad(table[N_FULL_BLK * 128:].T, ((0, 0), (0, 128 - TAIL)))
    tab2 = _build_relayout()(tabT, tailp)
    out3 = _build_gather()(valsT, tab2)
    return out3.transpose(2, 0, 1)


# speedup vs baseline: 1.3193x; 1.3193x over previous
import functools

import jax
import jax.numpy as jnp
from jax import lax
from jax.experimental import pallas as pl
from jax.experimental.pallas import tpu as pltpu
from jax.experimental.pallas import tpu_sc as plsc

DIM = 64
SCALE = 8.0
NC = 2
NS = 16
NW = NC * NS
LANES = 16
CHUNK = 128
FIRES = 4
STEP = CHUNK * FIRES


@functools.lru_cache(maxsize=None)
def _build(B):
    assert B % (NW * STEP) == 0
    b_per_w = B // NW
    n_steps = b_per_w // STEP
    n_pairs = (n_steps + 1) // 2
    mesh = plsc.VectorSubcoreMesh(
        core_axis_name="c", subcore_axis_name="s", num_cores=NC, num_subcores=NS
    )

    @functools.partial(
        pl.kernel,
        out_type=jax.ShapeDtypeStruct((B, DIM), jnp.float32),
        mesh=mesh,
        scratch_types=[
            pltpu.VMEM((FIRES, CHUNK), jnp.int32),
            pltpu.VMEM((FIRES, CHUNK), jnp.int32),
            pltpu.VMEM((STEP, DIM), jnp.float32),
            pltpu.VMEM((STEP, DIM), jnp.float32),
            pltpu.SemaphoreType.DMA,
            pltpu.SemaphoreType.DMA,
            pltpu.SemaphoreType.DMA,
            pltpu.SemaphoreType.DMA,
        ],
        compiler_params=pltpu.CompilerParams(
            use_tc_tiling_on_sc=False, needs_layout_passes=False
        ),
    )
    def emb(idx_hbm, table_hbm, out_hbm, i0, i1, r0, r1, g0, g1, o0, o1):
        wid = lax.axis_index("s") * NC + lax.axis_index("c")
        ibufs, rbufs = (i0, i1), (r0, r1)
        gsems, osems = (g0, g1), (o0, o1)
        idx_row0 = wid * (b_per_w // CHUNK)
        base = wid * b_per_w

        def start_step(t, p):
            @pl.when(t < n_steps)
            def _():
                pltpu.sync_copy(
                    idx_hbm.at[pl.ds(idx_row0 + t * FIRES, FIRES)], ibufs[p]
                )
                for j in range(FIRES):
                    pltpu.async_copy(
                        table_hbm.at[ibufs[p].at[j]],
                        rbufs[p].at[pl.ds(j * CHUNK, CHUNK)],
                        gsems[p],
                    )

        start_step(0, 0)

        def pair(t2, c):
            for p in (0, 1):
                t = t2 * 2 + p

                @pl.when(t < n_steps)
                def _():
                    @pl.when(t >= 1)
                    def _w():
                        pltpu.make_async_copy(
                            rbufs[1 - p], out_hbm.at[pl.ds(0, STEP)], osems[1 - p]
                        ).wait()

                    start_step(t + 1, 1 - p)

                    for j in range(FIRES):
                        pltpu.make_async_copy(
                            table_hbm.at[ibufs[p].at[j]],
                            rbufs[p].at[pl.ds(j * CHUNK, CHUNK)],
                            gsems[p],
                        ).wait()

                    @plsc.parallel_loop(0, STEP, 1, unroll=4)
                    def _scale(i):
                        for q in range(DIM // LANES):
                            sl = pl.ds(q * LANES, LANES)
                            rbufs[p][i, sl] = rbufs[p][i, sl] * SCALE

                    pltpu.async_copy(
                        rbufs[p], out_hbm.at[pl.ds(base + t * STEP, STEP)], osems[p]
                    )

            return c

        lax.fori_loop(0, n_pairs, pair, 0)
        last_p = (n_steps - 1) % 2
        pltpu.make_async_copy(
            rbufs[last_p], out_hbm.at[pl.ds(0, STEP)], osems[last_p]
        ).wait()

    return emb


def kernel(values, table):
    B = values.size
    idx2d = values.reshape(B // CHUNK, CHUNK)
    out = _build(B)(idx2d, table)
    return out.reshape(*values.shape, DIM)
